# baseline (speedup 1.0000x reference)
import functools

import jax
import jax.numpy as jnp
from jax import lax
from jax.experimental import pallas as pl
from jax.experimental.pallas import tpu as pltpu
from jax.experimental.pallas import tpu_sc as plsc

_B = 16
_C = 84
_N = 20000
_K = 100
_BLK = 2048
_NCHUNK = _N // 16
_IMIN = -2147483648
_ROW = 8
_CAP = 512
_NS = _CAP // 16


def _stage1_body(x_ref, key_ref, cat_ref):
    v = x_ref[...]
    ch = lax.broadcasted_iota(jnp.int32, v.shape, 1)
    sv = jnp.where(ch >= 4, v, -jnp.inf)
    m = jnp.max(sv, axis=1)
    cand = jnp.where(sv == m[:, None, :], ch - 4, _C)
    cat = jnp.min(cand, axis=1)
    mb = lax.bitcast_convert_type(m, jnp.int32)
    key_ref[...] = jnp.where(mb < 0, mb ^ 0x7FFFFFFF, mb)
    cat_ref[...] = cat


def _stage1(x):
    grid = (pl.cdiv(_N, _BLK),)
    return pl.pallas_call(
        _stage1_body,
        grid=grid,
        in_specs=[pl.BlockSpec((_B, _C, _BLK), lambda i: (0, 0, i))],
        out_specs=[
            pl.BlockSpec((_B, _BLK), lambda i: (0, i)),
            pl.BlockSpec((_B, _BLK), lambda i: (0, i)),
        ],
        out_shape=[
            jax.ShapeDtypeStruct((_B, _N), jnp.int32),
            jax.ShapeDtypeStruct((_B, _N), jnp.int32),
        ],
    )(x)


def _make_stage2():
    mesh = plsc.VectorSubcoreMesh(core_axis_name="c", subcore_axis_name="s",
                                  num_cores=2, num_subcores=16)

    @functools.partial(
        pl.kernel,
        out_type=jax.ShapeDtypeStruct((_B * _K * _ROW,), jnp.float32),
        mesh=mesh,
        scratch_types=[
            pltpu.VMEM((_N,), jnp.int32),
            pltpu.VMEM((_N,), jnp.int32),
            pltpu.VMEM((4, _N), jnp.float32),
            pltpu.VMEM((4096,), jnp.int32),
            pltpu.VMEM((128,), jnp.int32),
            pltpu.VMEM((128,), jnp.int32),
            pltpu.VMEM((128,), jnp.int32),
            pltpu.VMEM((_CAP + 16,), jnp.int32),
            pltpu.VMEM((_CAP + 16,), jnp.int32),
            pltpu.VMEM((224,), jnp.int32),
            pltpu.VMEM((112,), jnp.int32),
            pltpu.VMEM((112,), jnp.int32),
            pltpu.VMEM((_K * _ROW + 96,), jnp.float32),
            pltpu.VMEM((16,), jnp.float32),
        ],
        compiler_params=pltpu.CompilerParams(needs_layout_passes=False),
    )
    def stage2(key_hbm, cat_hbm, x_hbm, cm_hbm, out_hbm,
               keys_v, cat_v, boxes_v, hist_v, gt_v, eq_v, hi_v, mk_v, mi_v,
               cand_v, ckey_v, rpos_v, outv, cm_v):
        cid = lax.axis_index("c")
        sid = lax.axis_index("s")
        b = cid * 8 + sid

        @pl.when(sid < 8)
        def _body():
            lane = lax.broadcasted_iota(jnp.int32, (16,), 0)
            zeros = jnp.zeros((16,), jnp.int32)
            ones = jnp.ones((16,), jnp.int32)

            pltpu.sync_copy(key_hbm.at[b], keys_v)
            pltpu.sync_copy(cat_hbm.at[b], cat_v)
            pltpu.sync_copy(x_hbm.at[b, pl.ds(0, 4), :], boxes_v)
            pltpu.sync_copy(cm_hbm, cm_v)

            def _zero_hist():
                @pl.loop(0, 256)
                def _z(j):
                    hist_v[pl.ds(j * 16, 16)] = zeros

            def _scan256(need_val):
                def _s(j, carry):
                    above, bsel, found = carry
                    cnt = jnp.sum(hist_v[pl.ds((255 - j) * 16, 16)])
                    hit = jnp.logical_and(
                        jnp.logical_not(found), above + cnt >= need_val)
                    bsel = jnp.where(hit, 255 - j, bsel)
                    above = jnp.where(found | hit, above, above + cnt)
                    return above, bsel, found | hit

                return lax.fori_loop(
                    0, 256, _s,
                    (jnp.int32(0), jnp.int32(0), jnp.bool_(False)))

            _zero_hist()

            @pl.loop(0, _NCHUNK)
            def _h0(i):
                k = keys_v[pl.ds(i * 16, 16)]
                plsc.addupdate_scatter(
                    hist_v, [((k >> 24) + 128) * 16 + lane], ones)

            above0, b0, _ = _scan256(jnp.int32(_K))
            need1 = jnp.int32(_K) - above0

            _zero_hist()

            @pl.loop(0, _NCHUNK)
            def _h1(i):
                k = keys_v[pl.ds(i * 16, 16)]
                mask = ((k >> 24) + 128) == b0
                plsc.addupdate_scatter(
                    hist_v, [((k >> 16) & 0xFF) * 16 + lane], ones, mask=mask)

            above1, b1, _ = _scan256(need1)
            need2 = need1 - above1
            c16 = ((b0 - 128) << 8) | b1

            @pl.loop(0, 8)
            def _zb(j):
                gt_v[pl.ds(j * 16, 16)] = zeros
                eq_v[pl.ds(j * 16, 16)] = zeros
                hi_v[pl.ds(j * 16, 16)] = zeros

            @pl.loop(0, 14)
            def _zc(j):
                cand_v[pl.ds(j * 16, 16)] = zeros

            @pl.loop(0, 7)
            def _zr(j):
                rpos_v[pl.ds(j * 16, 16)] = zeros

            def _comp0(i, carry):
                phi, pmt, pms = carry
                k = keys_v[pl.ds(i * 16, 16)]
                idxv = lane + i * 16
                t16 = k >> 16
                mhi = t16 > c16
                plsc.store_compressed(hi_v.at[pl.ds(phi, 16)], idxv, mask=mhi)
                meq = t16 == c16
                rank = plsc.cumsum(jnp.where(meq, 1, 0))
                mtk = meq & ((pms + rank) <= _CAP)
                plsc.store_compressed(mk_v.at[pl.ds(pms, 16)], k, mask=mtk)
                plsc.store_compressed(mi_v.at[pl.ds(pms, 16)], idxv, mask=mtk)
                return (phi + jnp.sum(mhi.astype(jnp.int32)),
                        pmt + jnp.sum(meq.astype(jnp.int32)),
                        pms + jnp.sum(mtk.astype(jnp.int32)))

            phi, pmt, pms = lax.fori_loop(
                0, _NCHUNK, _comp0,
                (jnp.int32(0), jnp.int32(0), jnp.int32(0)))

            small = pmt <= _CAP

            @pl.when(small)
            def _small_path():
                _zero_hist()

                @pl.loop(0, _NS)
                def _h2(i):
                    k = mk_v[pl.ds(i * 16, 16)]
                    mask = (lane + i * 16) < pms
                    plsc.addupdate_scatter(
                        hist_v, [((k >> 8) & 0xFF) * 16 + lane], ones,
                        mask=mask)

                above2, b2, _ = _scan256(need2)
                need3 = need2 - above2

                _zero_hist()

                @pl.loop(0, _NS)
                def _h3(i):
                    k = mk_v[pl.ds(i * 16, 16)]
                    mask = (((lane + i * 16) < pms)
                            & (((k >> 8) & 0xFF) == b2))
                    plsc.addupdate_scatter(
                        hist_v, [(k & 0xFF) * 16 + lane], ones, mask=mask)

                above3, b3, _ = _scan256(need3)
                need4 = need3 - above3
                thr = (c16 << 16) | (b2 << 8) | b3
                count_gt = jnp.int32(_K) - need4

                def _compf(i, carry):
                    pgt, peq = carry
                    k = mk_v[pl.ds(i * 16, 16)]
                    iv = mi_v[pl.ds(i * 16, 16)]
                    valid = (lane + i * 16) < pms
                    mgt = valid & (k > thr)
                    plsc.store_compressed(
                        gt_v.at[pl.ds(pgt, 16)], iv, mask=mgt)
                    meq2 = valid & (k == thr)
                    rank = plsc.cumsum(jnp.where(meq2, 1, 0))
                    mtk2 = meq2 & ((peq + rank) <= need4)
                    plsc.store_compressed(
                        eq_v.at[pl.ds(peq, 16)], iv, mask=mtk2)
                    return (pgt + jnp.sum(mgt.astype(jnp.int32)),
                            peq + jnp.sum(mtk2.astype(jnp.int32)))

                lax.fori_loop(0, _NS, _compf, (jnp.int32(0), jnp.int32(0)))

                @pl.loop(0, 7)
                def _c1(j):
                    cand_v[pl.ds(j * 16, 16)] = hi_v[pl.ds(j * 16, 16)]

                @pl.loop(0, 7)
                def _c2(j):
                    cand_v[pl.ds(phi + j * 16, 16)] = gt_v[pl.ds(j * 16, 16)]

                @pl.loop(0, 7)
                def _c3(j):
                    cand_v[pl.ds(count_gt + j * 16, 16)] = (
                        eq_v[pl.ds(j * 16, 16)])

            @pl.when(jnp.logical_not(small))
            def _full_path():
                _zero_hist()

                @pl.loop(0, _NCHUNK)
                def _h2f(i):
                    k = keys_v[pl.ds(i * 16, 16)]
                    mask = (k >> 16) == c16
                    plsc.addupdate_scatter(
                        hist_v, [((k >> 8) & 0xFF) * 16 + lane], ones,
                        mask=mask)

                above2, b2, _ = _scan256(need2)
                need3 = need2 - above2

                _zero_hist()

                @pl.loop(0, _NCHUNK)
                def _h3f(i):
                    k = keys_v[pl.ds(i * 16, 16)]
                    mask = (((k >> 16) == c16)
                            & (((k >> 8) & 0xFF) == b2))
                    plsc.addupdate_scatter(
                        hist_v, [(k & 0xFF) * 16 + lane], ones, mask=mask)

                above3, b3, _ = _scan256(need3)
                need4 = need3 - above3
                thr = (c16 << 16) | (b2 << 8) | b3
                count_gt = jnp.int32(_K) - need4

                def _compff(i, carry):
                    pgt, peq = carry
                    k = keys_v[pl.ds(i * 16, 16)]
                    idxv = lane + i * 16
                    mgt = k > thr
                    plsc.store_compressed(
                        gt_v.at[pl.ds(pgt, 16)], idxv, mask=mgt)
                    meq2 = k == thr
                    rank = plsc.cumsum(jnp.where(meq2, 1, 0))
                    mtk2 = meq2 & ((peq + rank) <= need4)
                    plsc.store_compressed(
                        eq_v.at[pl.ds(peq, 16)], idxv, mask=mtk2)
                    return (pgt + jnp.sum(mgt.astype(jnp.int32)),
                            peq + jnp.sum(mtk2.astype(jnp.int32)))

                lax.fori_loop(0, _NCHUNK, _compff,
                              (jnp.int32(0), jnp.int32(0)))

                @pl.loop(0, 7)
                def _c1f(j):
                    cand_v[pl.ds(j * 16, 16)] = gt_v[pl.ds(j * 16, 16)]

                @pl.loop(0, 7)
                def _c2f(j):
                    cand_v[pl.ds(count_gt + j * 16, 16)] = (
                        eq_v[pl.ds(j * 16, 16)])

            @pl.loop(0, 7)
            def _ckeys(j):
                ci = cand_v[pl.ds(j * 16, 16)]
                kk = plsc.load_gather(keys_v, [ci])
                pos = lane + j * 16
                ckey_v[pl.ds(j * 16, 16)] = jnp.where(pos < _K, kk, _IMIN)

            mask0 = lane == 0

            def _rank(r, carry):
                mrun = ckey_v[pl.ds(0, 16)]
                for j in range(1, 7):
                    mrun = jnp.maximum(mrun, ckey_v[pl.ds(j * 16, 16)])
                mx = jnp.max(mrun)
                pos = jnp.int32(10000)
                for j in range(7):
                    f = jnp.min(plsc.all_reduce_ffs(
                        ckey_v[pl.ds(j * 16, 16)] == mx))
                    pos = jnp.where(f < 16, jnp.minimum(pos, j * 16 + f), pos)
                plsc.store_scatter(rpos_v, [jnp.broadcast_to(r, (16,))],
                                   jnp.broadcast_to(pos, (16,)), mask=mask0)
                plsc.store_scatter(ckey_v, [jnp.broadcast_to(pos, (16,))],
                                   jnp.full((16,), _IMIN, jnp.int32),
                                   mask=mask0)
                return carry

            lax.fori_loop(0, _K, _rank, jnp.int32(0))

            cmvec = cm_v[pl.ds(0, 16)]
            cm = [cmvec[i] for i in range(16)]
            bf = lax.convert_element_type(b, jnp.float32)
            bfv = jnp.broadcast_to(bf, (16,))

            @pl.loop(0, 7)
            def _emit(j):
                pos = rpos_v[pl.ds(j * 16, 16)]
                oi = plsc.load_gather(cand_v, [pos])
                kk = plsc.load_gather(keys_v, [oi])
                sc = plsc.bitcast(
                    jnp.where(kk < 0, kk ^ 0x7FFFFFFF, kk), jnp.float32)
                ct = plsc.load_gather(cat_v, [oi]).astype(jnp.float32)
                cx = plsc.load_gather(boxes_v, [zeros, oi])
                cy = plsc.load_gather(boxes_v, [zeros + 1, oi])
                ww = plsc.load_gather(boxes_v, [zeros + 2, oi])
                hh = plsc.load_gather(boxes_v, [zeros + 3, oi])
                o0 = cx * cm[0] + cy * cm[4] + ww * cm[8] + hh * cm[12]
                o1 = cx * cm[1] + cy * cm[5] + ww * cm[9] + hh * cm[13]
                o2 = cx * cm[2] + cy * cm[6] + ww * cm[10] + hh * cm[14]
                o3 = cx * cm[3] + cy * cm[7] + ww * cm[11] + hh * cm[15]
                base = lane * _ROW + j * 16 * _ROW
                plsc.store_scatter(outv, [base + 0], bfv)
                plsc.store_scatter(outv, [base + 1], o0)
                plsc.store_scatter(outv, [base + 2], o1)
                plsc.store_scatter(outv, [base + 3], o2)
                plsc.store_scatter(outv, [base + 4], o3)
                plsc.store_scatter(outv, [base + 5], ct)
                plsc.store_scatter(outv, [base + 6], sc)

            pltpu.sync_copy(outv.at[pl.ds(0, _K * _ROW)],
                            out_hbm.at[pl.ds(b * _K * _ROW, _K * _ROW)])

    return stage2


_stage2_kernel = _make_stage2()


def kernel(x, convert_matrix):
    keys, cat = _stage1(x)
    out = _stage2_kernel(keys, cat, x, convert_matrix.reshape(16))
    return out.reshape(_B * _K, _ROW)[:, :7]


# speedup vs baseline: 1.0684x; 1.0684x over previous
import functools

import jax
import jax.numpy as jnp
from jax import lax
from jax.experimental import pallas as pl
from jax.experimental.pallas import tpu as pltpu
from jax.experimental.pallas import tpu_sc as plsc

_B = 16
_C = 84
_N = 20000
_K = 100
_BLK = 2048
_NCHUNK = _N // 16
_IMIN = -2147483648
_ROW = 8
_CAP = 512
_NS = _CAP // 16


def _stage1_body(x_ref, key_ref, cat_ref):
    v = x_ref[...]
    ch = lax.broadcasted_iota(jnp.int32, v.shape, 1)
    sv = jnp.where(ch >= 4, v, -jnp.inf)
    m = jnp.max(sv, axis=1)
    cand = jnp.where(sv == m[:, None, :], ch - 4, _C)
    cat = jnp.min(cand, axis=1)
    mb = lax.bitcast_convert_type(m, jnp.int32)
    key_ref[...] = jnp.where(mb < 0, mb ^ 0x7FFFFFFF, mb)
    cat_ref[...] = cat


def _stage1(x):
    grid = (pl.cdiv(_N, _BLK),)
    return pl.pallas_call(
        _stage1_body,
        grid=grid,
        in_specs=[pl.BlockSpec((_B, _C, _BLK), lambda i: (0, 0, i))],
        out_specs=[
            pl.BlockSpec((_B, _BLK), lambda i: (0, i)),
            pl.BlockSpec((_B, _BLK), lambda i: (0, i)),
        ],
        out_shape=[
            jax.ShapeDtypeStruct((_B, _N), jnp.int32),
            jax.ShapeDtypeStruct((_B, _N), jnp.int32),
        ],
    )(x)


def _make_stage2():
    mesh = plsc.VectorSubcoreMesh(core_axis_name="c", subcore_axis_name="s",
                                  num_cores=2, num_subcores=16)

    @functools.partial(
        pl.kernel,
        out_type=jax.ShapeDtypeStruct((_B * _K * _ROW,), jnp.float32),
        mesh=mesh,
        scratch_types=[
            pltpu.VMEM((_N,), jnp.int32),
            pltpu.VMEM((_N,), jnp.int32),
            pltpu.VMEM((4, _N), jnp.float32),
            pltpu.VMEM((4096,), jnp.int32),
            pltpu.VMEM((128,), jnp.int32),
            pltpu.VMEM((128,), jnp.int32),
            pltpu.VMEM((128,), jnp.int32),
            pltpu.VMEM((_CAP + 16,), jnp.int32),
            pltpu.VMEM((_CAP + 16,), jnp.int32),
            pltpu.VMEM((224,), jnp.int32),
            pltpu.VMEM((112,), jnp.int32),
            pltpu.VMEM((112,), jnp.int32),
            pltpu.VMEM((_K * _ROW + 96,), jnp.float32),
            pltpu.VMEM((16,), jnp.float32),
            pltpu.SemaphoreType.DMA,
            pltpu.SemaphoreType.DMA,
        ],
        compiler_params=pltpu.CompilerParams(needs_layout_passes=False),
    )
    def stage2(key_hbm, cat_hbm, x_hbm, cm_hbm, out_hbm,
               keys_v, cat_v, boxes_v, hist_v, gt_v, eq_v, hi_v, mk_v, mi_v,
               cand_v, ckey_v, rpos_v, outv, cm_v, sem1, sem2):
        cid = lax.axis_index("c")
        sid = lax.axis_index("s")
        b = cid * 8 + sid

        @pl.when(sid < 8)
        def _body():
            lane = lax.broadcasted_iota(jnp.int32, (16,), 0)
            zeros = jnp.zeros((16,), jnp.int32)
            ones = jnp.ones((16,), jnp.int32)

            cat_dma = pltpu.async_copy(cat_hbm.at[b], cat_v, sem1)
            box_dma = pltpu.async_copy(x_hbm.at[b, pl.ds(0, 4), :],
                                       boxes_v, sem2)
            pltpu.sync_copy(key_hbm.at[b], keys_v)
            pltpu.sync_copy(cm_hbm, cm_v)

            def _zero_hist():
                @pl.loop(0, 256, unroll=8)
                def _z(j):
                    hist_v[pl.ds(j * 16, 16)] = zeros

            def _scan256(need_val):
                def _cond(carry):
                    j, above, bsel, found = carry
                    return jnp.logical_and(jnp.logical_not(found), j < 256)

                def _s(carry):
                    j, above, bsel, found = carry
                    cnt = jnp.sum(hist_v[pl.ds((255 - j) * 16, 16)])
                    hit = above + cnt >= need_val
                    bsel = jnp.where(hit, 255 - j, bsel)
                    above = jnp.where(hit, above, above + cnt)
                    return j + 1, above, bsel, hit

                _, above, bsel, _ = lax.while_loop(
                    _cond, _s,
                    (jnp.int32(0), jnp.int32(0), jnp.int32(0),
                     jnp.bool_(False)))
                return above, bsel, None

            _zero_hist()

            @pl.loop(0, _NCHUNK, unroll=5)
            def _h0(i):
                k = keys_v[pl.ds(i * 16, 16)]
                plsc.addupdate_scatter(
                    hist_v, [((k >> 24) + 128) * 16 + lane], ones)

            above0, b0, _ = _scan256(jnp.int32(_K))
            need1 = jnp.int32(_K) - above0

            _zero_hist()

            @pl.loop(0, _NCHUNK, unroll=5)
            def _h1(i):
                k = keys_v[pl.ds(i * 16, 16)]
                mask = ((k >> 24) + 128) == b0
                plsc.addupdate_scatter(
                    hist_v, [((k >> 16) & 0xFF) * 16 + lane], ones, mask=mask)

            above1, b1, _ = _scan256(need1)
            need2 = need1 - above1
            c16 = ((b0 - 128) << 8) | b1

            @pl.loop(0, 8)
            def _zb(j):
                gt_v[pl.ds(j * 16, 16)] = zeros
                eq_v[pl.ds(j * 16, 16)] = zeros
                hi_v[pl.ds(j * 16, 16)] = zeros

            @pl.loop(0, 14)
            def _zc(j):
                cand_v[pl.ds(j * 16, 16)] = zeros

            @pl.loop(0, 7)
            def _zr(j):
                rpos_v[pl.ds(j * 16, 16)] = zeros

            def _comp0(i, carry):
                phi, pmt, pms = carry
                k = keys_v[pl.ds(i * 16, 16)]
                idxv = lane + i * 16
                t16 = k >> 16
                mhi = t16 > c16
                plsc.store_compressed(hi_v.at[pl.ds(phi, 16)], idxv, mask=mhi)
                meq = t16 == c16
                plsc.store_compressed(mk_v.at[pl.ds(pms, 16)], k, mask=meq)
                plsc.store_compressed(mi_v.at[pl.ds(pms, 16)], idxv, mask=meq)
                cnt = jnp.sum(meq.astype(jnp.int32))
                return (phi + jnp.sum(mhi.astype(jnp.int32)),
                        pmt + cnt,
                        jnp.minimum(pms + cnt, jnp.int32(_CAP)))

            phi, pmt, pms = lax.fori_loop(
                0, _NCHUNK, _comp0,
                (jnp.int32(0), jnp.int32(0), jnp.int32(0)))

            small = pmt <= _CAP

            @pl.when(small)
            def _small_path():
                _zero_hist()

                @pl.loop(0, _NS)
                def _h2(i):
                    k = mk_v[pl.ds(i * 16, 16)]
                    mask = (lane + i * 16) < pms
                    plsc.addupdate_scatter(
                        hist_v, [((k >> 8) & 0xFF) * 16 + lane], ones,
                        mask=mask)

                above2, b2, _ = _scan256(need2)
                need3 = need2 - above2

                _zero_hist()

                @pl.loop(0, _NS)
                def _h3(i):
                    k = mk_v[pl.ds(i * 16, 16)]
                    mask = (((lane + i * 16) < pms)
                            & (((k >> 8) & 0xFF) == b2))
                    plsc.addupdate_scatter(
                        hist_v, [(k & 0xFF) * 16 + lane], ones, mask=mask)

                above3, b3, _ = _scan256(need3)
                need4 = need3 - above3
                thr = (c16 << 16) | (b2 << 8) | b3
                count_gt = jnp.int32(_K) - need4

                def _compf(i, carry):
                    pgt, peq = carry
                    k = mk_v[pl.ds(i * 16, 16)]
                    iv = mi_v[pl.ds(i * 16, 16)]
                    valid = (lane + i * 16) < pms
                    mgt = valid & (k > thr)
                    plsc.store_compressed(
                        gt_v.at[pl.ds(pgt, 16)], iv, mask=mgt)
                    meq2 = valid & (k == thr)
                    rank = plsc.cumsum(jnp.where(meq2, 1, 0))
                    mtk2 = meq2 & ((peq + rank) <= need4)
                    plsc.store_compressed(
                        eq_v.at[pl.ds(peq, 16)], iv, mask=mtk2)
                    return (pgt + jnp.sum(mgt.astype(jnp.int32)),
                            peq + jnp.sum(mtk2.astype(jnp.int32)))

                lax.fori_loop(0, _NS, _compf, (jnp.int32(0), jnp.int32(0)))

                @pl.loop(0, 7)
                def _c1(j):
                    cand_v[pl.ds(j * 16, 16)] = hi_v[pl.ds(j * 16, 16)]

                @pl.loop(0, 7)
                def _c2(j):
                    cand_v[pl.ds(phi + j * 16, 16)] = gt_v[pl.ds(j * 16, 16)]

                @pl.loop(0, 7)
                def _c3(j):
                    cand_v[pl.ds(count_gt + j * 16, 16)] = (
                        eq_v[pl.ds(j * 16, 16)])

            @pl.when(jnp.logical_not(small))
            def _full_path():
                _zero_hist()

                @pl.loop(0, _NCHUNK)
                def _h2f(i):
                    k = keys_v[pl.ds(i * 16, 16)]
                    mask = (k >> 16) == c16
                    plsc.addupdate_scatter(
                        hist_v, [((k >> 8) & 0xFF) * 16 + lane], ones,
                        mask=mask)

                above2, b2, _ = _scan256(need2)
                need3 = need2 - above2

                _zero_hist()

                @pl.loop(0, _NCHUNK)
                def _h3f(i):
                    k = keys_v[pl.ds(i * 16, 16)]
                    mask = (((k >> 16) == c16)
                            & (((k >> 8) & 0xFF) == b2))
                    plsc.addupdate_scatter(
                        hist_v, [(k & 0xFF) * 16 + lane], ones, mask=mask)

                above3, b3, _ = _scan256(need3)
                need4 = need3 - above3
                thr = (c16 << 16) | (b2 << 8) | b3
                count_gt = jnp.int32(_K) - need4

                def _compff(i, carry):
                    pgt, peq = carry
                    k = keys_v[pl.ds(i * 16, 16)]
                    idxv = lane + i * 16
                    mgt = k > thr
                    plsc.store_compressed(
                        gt_v.at[pl.ds(pgt, 16)], idxv, mask=mgt)
                    meq2 = k == thr
                    rank = plsc.cumsum(jnp.where(meq2, 1, 0))
                    mtk2 = meq2 & ((peq + rank) <= need4)
                    plsc.store_compressed(
                        eq_v.at[pl.ds(peq, 16)], idxv, mask=mtk2)
                    return (pgt + jnp.sum(mgt.astype(jnp.int32)),
                            peq + jnp.sum(mtk2.astype(jnp.int32)))

                lax.fori_loop(0, _NCHUNK, _compff,
                              (jnp.int32(0), jnp.int32(0)))

                @pl.loop(0, 7)
                def _c1f(j):
                    cand_v[pl.ds(j * 16, 16)] = gt_v[pl.ds(j * 16, 16)]

                @pl.loop(0, 7)
                def _c2f(j):
                    cand_v[pl.ds(count_gt + j * 16, 16)] = (
                        eq_v[pl.ds(j * 16, 16)])

            @pl.loop(0, 7)
            def _ckeys(j):
                ci = cand_v[pl.ds(j * 16, 16)]
                kk = plsc.load_gather(keys_v, [ci])
                pos = lane + j * 16
                ckey_v[pl.ds(j * 16, 16)] = jnp.where(pos < _K, kk, _IMIN)

            mask0 = lane == 0

            def _rank(r, carry):
                mrun = ckey_v[pl.ds(0, 16)]
                for j in range(1, 7):
                    mrun = jnp.maximum(mrun, ckey_v[pl.ds(j * 16, 16)])
                mx = jnp.max(mrun)
                pos = jnp.int32(10000)
                for j in range(7):
                    f = plsc.all_reduce_ffs(
                        ckey_v[pl.ds(j * 16, 16)] == mx)[0]
                    pos = jnp.where(f < 16, jnp.minimum(pos, j * 16 + f), pos)
                plsc.store_scatter(rpos_v, [jnp.broadcast_to(r, (16,))],
                                   jnp.broadcast_to(pos, (16,)), mask=mask0)
                plsc.store_scatter(ckey_v, [jnp.broadcast_to(pos, (16,))],
                                   jnp.full((16,), _IMIN, jnp.int32),
                                   mask=mask0)
                return carry

            lax.fori_loop(0, _K, _rank, jnp.int32(0))

            cat_dma.wait()
            box_dma.wait()
            cmvec = cm_v[pl.ds(0, 16)]
            cm = [cmvec[i] for i in range(16)]
            bf = lax.convert_element_type(b, jnp.float32)
            bfv = jnp.broadcast_to(bf, (16,))

            @pl.loop(0, 7)
            def _emit(j):
                pos = rpos_v[pl.ds(j * 16, 16)]
                oi = plsc.load_gather(cand_v, [pos])
                kk = plsc.load_gather(keys_v, [oi])
                sc = plsc.bitcast(
                    jnp.where(kk < 0, kk ^ 0x7FFFFFFF, kk), jnp.float32)
                ct = plsc.load_gather(cat_v, [oi]).astype(jnp.float32)
                cx = plsc.load_gather(boxes_v, [zeros, oi])
                cy = plsc.load_gather(boxes_v, [zeros + 1, oi])
                ww = plsc.load_gather(boxes_v, [zeros + 2, oi])
                hh = plsc.load_gather(boxes_v, [zeros + 3, oi])
                o0 = cx * cm[0] + cy * cm[4] + ww * cm[8] + hh * cm[12]
                o1 = cx * cm[1] + cy * cm[5] + ww * cm[9] + hh * cm[13]
                o2 = cx * cm[2] + cy * cm[6] + ww * cm[10] + hh * cm[14]
                o3 = cx * cm[3] + cy * cm[7] + ww * cm[11] + hh * cm[15]
                base = lane * _ROW + j * 16 * _ROW
                plsc.store_scatter(outv, [base + 0], bfv)
                plsc.store_scatter(outv, [base + 1], o0)
                plsc.store_scatter(outv, [base + 2], o1)
                plsc.store_scatter(outv, [base + 3], o2)
                plsc.store_scatter(outv, [base + 4], o3)
                plsc.store_scatter(outv, [base + 5], ct)
                plsc.store_scatter(outv, [base + 6], sc)

            pltpu.sync_copy(outv.at[pl.ds(0, _K * _ROW)],
                            out_hbm.at[pl.ds(b * _K * _ROW, _K * _ROW)])

    return stage2


_stage2_kernel = _make_stage2()


def kernel(x, convert_matrix):
    keys, cat = _stage1(x)
    out = _stage2_kernel(keys, cat, x, convert_matrix.reshape(16))
    return out.reshape(_B * _K, _ROW)[:, :7]
